# speedup vs baseline: 8.6415x; 8.6415x over previous
import functools

import jax
import jax.numpy as jnp
from jax import lax
from jax.experimental import pallas as pl
from jax.experimental.pallas import tpu as pltpu
from jax.experimental.pallas import tpu_sc as plsc

N_REAL = 10000
D = 128
E_REAL = 320000

NC = 2
NS = 16
NW = NC * NS
CH = 128

N_PAD = 10240
ROWS_PER_TILE = N_PAD // NS
E_PAD = 2560 * CH
CHUNKS_PER_W = (E_PAD // CH) // NW

_mesh = plsc.VectorSubcoreMesh(
    core_axis_name="c", subcore_axis_name="s", num_cores=NC, num_subcores=NS)


def _worker_id():
    cid = lax.axis_index("c")
    sid = lax.axis_index("s")
    return cid, sid, sid * NC + cid


@functools.partial(
    pl.kernel,
    out_type=jax.ShapeDtypeStruct((NC, N_PAD), jnp.float32),
    mesh=_mesh,
    scratch_types=[
        pltpu.VMEM((1, CH), jnp.int32),
        pltpu.VMEM((CH,), jnp.float32),
        pltpu.VMEM((ROWS_PER_TILE,), jnp.float32),
        pltpu.VMEM_SHARED((N_PAD,), jnp.float32),
    ],
)
def _deg_kernel(dst_hbm, out_hbm, didx, ones_v, zbuf, hist_sh):
    cid, sid, w = _worker_id()
    one = jnp.full((16,), 1.0, dtype=jnp.float32)
    zero = jnp.zeros((16,), dtype=jnp.float32)
    for k in range(CH // 16):
        ones_v[pl.ds(k * 16, 16)] = one
    for k in range(ROWS_PER_TILE // 16):
        zbuf[pl.ds(k * 16, 16)] = zero
    pltpu.sync_copy(zbuf, hist_sh.at[pl.ds(sid * ROWS_PER_TILE, ROWS_PER_TILE)])
    plsc.subcore_barrier()

    def body(j, carry):
        ch = w * CHUNKS_PER_W + j
        pltpu.sync_copy(dst_hbm.at[ch], didx.at[0])
        pltpu.sync_copy(ones_v, hist_sh.at[didx.at[0]], add=True)
        return carry

    lax.fori_loop(0, CHUNKS_PER_W, body, 0)
    plsc.subcore_barrier()
    sl = pl.ds(sid * ROWS_PER_TILE, ROWS_PER_TILE)
    pltpu.sync_copy(hist_sh.at[sl], zbuf)
    pltpu.sync_copy(zbuf, out_hbm.at[cid, sl])


@functools.partial(
    pl.kernel,
    out_type=jax.ShapeDtypeStruct((NC, N_PAD, D), jnp.float32),
    mesh=_mesh,
    scratch_types=[
        pltpu.VMEM((1, CH), jnp.int32),
        pltpu.VMEM((1, CH), jnp.int32),
        pltpu.VMEM((CH, D), jnp.float32),
        pltpu.VMEM_SHARED((N_PAD, D), jnp.float32),
    ],
)
def _scatter_kernel(g_hbm, src_hbm, dst_hbm, out_hbm, sidx, didx, rows, acc_sh):
    cid, sid, w = _worker_id()
    zero = jnp.zeros((16,), dtype=jnp.float32)

    def zbody(i, carry):
        for k in range(D // 16):
            rows[i, pl.ds(k * 16, 16)] = zero
        return carry

    lax.fori_loop(0, CH, zbody, 0)
    base = sid * ROWS_PER_TILE
    for r in range(ROWS_PER_TILE // CH):
        pltpu.sync_copy(rows, acc_sh.at[pl.ds(base + r * CH, CH), :])
    plsc.subcore_barrier()

    def body(j, carry):
        ch = w * CHUNKS_PER_W + j
        pltpu.sync_copy(src_hbm.at[ch], sidx.at[0])
        pltpu.sync_copy(dst_hbm.at[ch], didx.at[0])
        pltpu.sync_copy(g_hbm.at[sidx.at[0]], rows)
        pltpu.sync_copy(rows, acc_sh.at[didx.at[0]], add=True)
        return carry

    lax.fori_loop(0, CHUNKS_PER_W, body, 0)
    plsc.subcore_barrier()
    for r in range(ROWS_PER_TILE // CH):
        sl = pl.ds(base + r * CH, CH)
        pltpu.sync_copy(acc_sh.at[sl, :], rows)
        pltpu.sync_copy(rows, out_hbm.at[cid, sl, :])


BLK = 1024


def _tc1_body(deg_ref, x_ref, w_ref, g_ref, dinv_ref):
    i = pl.program_id(0)
    deg = deg_ref[:, 0:1] + deg_ref[:, 1:2] + 1.0
    dinv = lax.rsqrt(deg)
    rows = lax.broadcasted_iota(jnp.int32, (BLK, 1), 0) + i * BLK
    dinv = jnp.where(rows < N_REAL, dinv, 0.0)
    g = jnp.dot(x_ref[...], w_ref[...], preferred_element_type=jnp.float32)
    g_ref[...] = g * dinv
    dinv_ref[...] = dinv


def _tc_mid_body(s_ref, g_ref, dinv_ref, b_ref, w_ref, g2_ref):
    s = s_ref[0] + s_ref[1] + g_ref[...]
    h = jnp.maximum(dinv_ref[...] * s + b_ref[...], 0.0)
    g2 = jnp.dot(h, w_ref[...], preferred_element_type=jnp.float32)
    g2_ref[...] = g2 * dinv_ref[...]


def _tc_out_body(s_ref, g_ref, dinv_ref, b_ref, w_ref, bo_ref, o_ref):
    s = s_ref[0] + s_ref[1] + g_ref[...]
    h = jnp.maximum(dinv_ref[...] * s + b_ref[...], 0.0)
    o_ref[...] = jnp.dot(h, w_ref[...],
                         preferred_element_type=jnp.float32) + bo_ref[...]


def _row_specs():
    s = pl.BlockSpec((2, BLK, D), lambda i: (0, i, 0))
    g = pl.BlockSpec((BLK, D), lambda i: (i, 0))
    v = pl.BlockSpec((BLK, 1), lambda i: (i, 0))
    w = pl.BlockSpec((D, D), lambda i: (0, 0))
    b = pl.BlockSpec((1, D), lambda i: (0, 0))
    return s, g, v, w, b


def kernel(x, edge_index, W1, b1, W2, b2, W_out, b_out):
    src = edge_index[0].astype(jnp.int32)
    dst = edge_index[1].astype(jnp.int32)
    pad_node = jnp.int32(N_PAD - 1)
    src2d = jnp.full((E_PAD,), pad_node).at[:E_REAL].set(src).reshape(-1, CH)
    dst2d = jnp.full((E_PAD,), pad_node).at[:E_REAL].set(dst).reshape(-1, CH)
    x_p = jnp.pad(x, ((0, N_PAD - N_REAL), (0, 0)))

    degp = _deg_kernel(dst2d)
    degp_t = jnp.swapaxes(degp, 0, 1)

    grid = N_PAD // BLK
    sspec, gspec, vspec, wspec, bspec = _row_specs()

    g1, dinv = pl.pallas_call(
        _tc1_body,
        grid=(grid,),
        in_specs=[pl.BlockSpec((BLK, 2), lambda i: (i, 0)), gspec, wspec],
        out_specs=[gspec, vspec],
        out_shape=[jax.ShapeDtypeStruct((N_PAD, D), jnp.float32),
                   jax.ShapeDtypeStruct((N_PAD, 1), jnp.float32)],
    )(degp_t, x_p, W1)

    s1 = _scatter_kernel(g1, src2d, dst2d)

    g2 = pl.pallas_call(
        _tc_mid_body,
        grid=(grid,),
        in_specs=[sspec, gspec, vspec, bspec, wspec],
        out_specs=gspec,
        out_shape=jax.ShapeDtypeStruct((N_PAD, D), jnp.float32),
    )(s1, g1, dinv, b1.reshape(1, D), W2)

    s2 = _scatter_kernel(g2, src2d, dst2d)

    out = pl.pallas_call(
        _tc_out_body,
        grid=(grid,),
        in_specs=[sspec, gspec, vspec, bspec,
                  pl.BlockSpec((D, 1), lambda i: (0, 0)),
                  pl.BlockSpec((1, 1), lambda i: (0, 0))],
        out_specs=vspec,
        out_shape=jax.ShapeDtypeStruct((N_PAD, 1), jnp.float32),
    )(s2, g2, dinv, b2.reshape(1, D), W_out, b_out.reshape(1, 1))

    return out[:N_REAL, 0]


# speedup vs baseline: 11.5572x; 1.3374x over previous
import functools

import jax
import jax.numpy as jnp
from jax import lax
from jax.experimental import pallas as pl
from jax.experimental.pallas import tpu as pltpu
from jax.experimental.pallas import tpu_sc as plsc

N_REAL = 10000
D = 128
E_REAL = 320000

NC = 2
NS = 16
NW = NC * NS
CH = 128

N_PAD = 10240
ROWS_PER_TILE = N_PAD // NS
E_PAD = 2560 * CH
CHUNKS_PER_W = (E_PAD // CH) // NW

_mesh = plsc.VectorSubcoreMesh(
    core_axis_name="c", subcore_axis_name="s", num_cores=NC, num_subcores=NS)


def _worker_id():
    cid = lax.axis_index("c")
    sid = lax.axis_index("s")
    return cid, sid, sid * NC + cid


@functools.partial(
    pl.kernel,
    out_type=jax.ShapeDtypeStruct((NC, N_PAD), jnp.float32),
    mesh=_mesh,
    scratch_types=[
        pltpu.VMEM((CHUNKS_PER_W, CH), jnp.int32),
        pltpu.VMEM((CH,), jnp.float32),
        pltpu.VMEM((ROWS_PER_TILE,), jnp.float32),
        pltpu.VMEM_SHARED((N_PAD,), jnp.float32),
        pltpu.SemaphoreType.DMA,
    ],
)
def _deg_kernel(dst_hbm, out_hbm, didx, ones_v, zbuf, hist_sh, sem):
    cid, sid, w = _worker_id()
    one = jnp.full((16,), 1.0, dtype=jnp.float32)
    zero = jnp.zeros((16,), dtype=jnp.float32)
    for k in range(CH // 16):
        ones_v[pl.ds(k * 16, 16)] = one
    for k in range(ROWS_PER_TILE // 16):
        zbuf[pl.ds(k * 16, 16)] = zero
    pltpu.sync_copy(dst_hbm.at[pl.ds(w * CHUNKS_PER_W, CHUNKS_PER_W), :], didx)
    pltpu.sync_copy(zbuf, hist_sh.at[pl.ds(sid * ROWS_PER_TILE, ROWS_PER_TILE)])
    plsc.subcore_barrier()

    K = 8

    def body(g, carry):
        ds_ = [pltpu.async_copy(ones_v, hist_sh.at[didx.at[g * K + b]], sem,
                                add=True) for b in range(K)]
        for d_ in ds_:
            d_.wait()
        return carry

    lax.fori_loop(0, CHUNKS_PER_W // K, body, 0)
    plsc.subcore_barrier()
    sl = pl.ds(sid * ROWS_PER_TILE, ROWS_PER_TILE)
    pltpu.sync_copy(hist_sh.at[sl], zbuf)
    pltpu.sync_copy(zbuf, out_hbm.at[cid, sl])


PASSES = 2
PASS_CH = CHUNKS_PER_W // PASSES


@functools.partial(
    pl.kernel,
    out_type=jax.ShapeDtypeStruct((NC, N_PAD, D), jnp.float32),
    mesh=_mesh,
    scratch_types=[
        pltpu.VMEM((PASS_CH, CH), jnp.int32),
        pltpu.VMEM((PASS_CH, CH), jnp.int32),
        pltpu.VMEM((2, CH, D), jnp.float32),
        pltpu.VMEM_SHARED((N_PAD, D), jnp.float32),
        pltpu.SemaphoreType.DMA,
        pltpu.SemaphoreType.DMA,
        pltpu.SemaphoreType.DMA,
        pltpu.SemaphoreType.DMA,
    ],
)
def _scatter_kernel(g_hbm, src_hbm, dst_hbm, out_hbm, sidx, didx, rows,
                    acc_sh, gsem0, gsem1, ssem0, ssem1):
    cid, sid, w = _worker_id()
    gsem = (gsem0, gsem1)
    ssem = (ssem0, ssem1)
    zero = jnp.zeros((16,), dtype=jnp.float32)

    def zbody(i, carry):
        for k in range(D // 16):
            rows[0, i, pl.ds(k * 16, 16)] = zero
        return carry

    lax.fori_loop(0, CH, zbody, 0)
    base = sid * ROWS_PER_TILE
    for r in range(ROWS_PER_TILE // CH):
        pltpu.sync_copy(rows.at[0], acc_sh.at[pl.ds(base + r * CH, CH), :])
    plsc.subcore_barrier()

    def gather(k, b):
        return pltpu.async_copy(g_hbm.at[sidx.at[k]], rows.at[b], gsem[b])

    def gather_wait(k, b):
        pltpu.make_async_copy(g_hbm.at[sidx.at[k]], rows.at[b],
                              gsem[b]).wait()

    def scatter(k, b):
        return pltpu.async_copy(rows.at[b], acc_sh.at[didx.at[k]], ssem[b],
                                add=True)

    def scatter_wait(k, b):
        pltpu.make_async_copy(rows.at[b], acc_sh.at[didx.at[k]],
                              ssem[b]).wait()

    for p in range(PASSES):
        off = w * CHUNKS_PER_W + p * PASS_CH
        pltpu.sync_copy(src_hbm.at[pl.ds(off, PASS_CH), :], sidx)
        pltpu.sync_copy(dst_hbm.at[pl.ds(off, PASS_CH), :], didx)
        gather(0, 0)

        def pair_body(t, carry):
            for u in range(2):
                k = 2 * t + u
                pu, nu = (u + 1) % 2, (u + 1) % 2

                @pl.when(k >= 1)
                def _():
                    scatter_wait(k - 1, pu)

                @pl.when(k + 1 < PASS_CH)
                def _():
                    gather(k + 1, nu)

                gather_wait(k, u)
                scatter(k, u)
            return carry

        lax.fori_loop(0, PASS_CH // 2, pair_body, 0)
        scatter_wait(PASS_CH - 1, 1)
    plsc.subcore_barrier()
    outs = []
    for r in range(ROWS_PER_TILE // CH):
        sl = pl.ds(base + r * CH, CH)
        outs.append(pltpu.async_copy(acc_sh.at[sl, :], out_hbm.at[cid, sl, :],
                                     gsem0))
    for d_ in outs:
        d_.wait()


BLK = 1024


def _tc1_body(deg_ref, x_ref, w_ref, g_ref, dinv_ref):
    i = pl.program_id(0)
    deg = deg_ref[:, 0:1] + deg_ref[:, 1:2] + 1.0
    dinv = lax.rsqrt(deg)
    rows = lax.broadcasted_iota(jnp.int32, (BLK, 1), 0) + i * BLK
    dinv = jnp.where(rows < N_REAL, dinv, 0.0)
    g = jnp.dot(x_ref[...], w_ref[...], preferred_element_type=jnp.float32)
    g_ref[...] = g * dinv
    dinv_ref[...] = dinv


def _tc_mid_body(s_ref, g_ref, dinv_ref, b_ref, w_ref, g2_ref):
    s = s_ref[0] + s_ref[1] + g_ref[...]
    h = jnp.maximum(dinv_ref[...] * s + b_ref[...], 0.0)
    g2 = jnp.dot(h, w_ref[...], preferred_element_type=jnp.float32)
    g2_ref[...] = g2 * dinv_ref[...]


def _tc_out_body(s_ref, g_ref, dinv_ref, b_ref, w_ref, bo_ref, o_ref):
    s = s_ref[0] + s_ref[1] + g_ref[...]
    h = jnp.maximum(dinv_ref[...] * s + b_ref[...], 0.0)
    o_ref[...] = jnp.dot(h, w_ref[...],
                         preferred_element_type=jnp.float32) + bo_ref[...]


def _row_specs():
    s = pl.BlockSpec((2, BLK, D), lambda i: (0, i, 0))
    g = pl.BlockSpec((BLK, D), lambda i: (i, 0))
    v = pl.BlockSpec((BLK, 1), lambda i: (i, 0))
    w = pl.BlockSpec((D, D), lambda i: (0, 0))
    b = pl.BlockSpec((1, D), lambda i: (0, 0))
    return s, g, v, w, b


def kernel(x, edge_index, W1, b1, W2, b2, W_out, b_out):
    src = edge_index[0].astype(jnp.int32)
    dst = edge_index[1].astype(jnp.int32)
    pad_node = jnp.int32(N_PAD - 1)
    src2d = jnp.full((E_PAD,), pad_node).at[:E_REAL].set(src).reshape(-1, CH)
    dst2d = jnp.full((E_PAD,), pad_node).at[:E_REAL].set(dst).reshape(-1, CH)
    x_p = jnp.pad(x, ((0, N_PAD - N_REAL), (0, 0)))

    degp = _deg_kernel(dst2d)
    degp_t = jnp.swapaxes(degp, 0, 1)

    grid = N_PAD // BLK
    sspec, gspec, vspec, wspec, bspec = _row_specs()

    g1, dinv = pl.pallas_call(
        _tc1_body,
        grid=(grid,),
        in_specs=[pl.BlockSpec((BLK, 2), lambda i: (i, 0)), gspec, wspec],
        out_specs=[gspec, vspec],
        out_shape=[jax.ShapeDtypeStruct((N_PAD, D), jnp.float32),
                   jax.ShapeDtypeStruct((N_PAD, 1), jnp.float32)],
    )(degp_t, x_p, W1)

    s1 = _scatter_kernel(g1, src2d, dst2d)

    g2 = pl.pallas_call(
        _tc_mid_body,
        grid=(grid,),
        in_specs=[sspec, gspec, vspec, bspec, wspec],
        out_specs=gspec,
        out_shape=jax.ShapeDtypeStruct((N_PAD, D), jnp.float32),
    )(s1, g1, dinv, b1.reshape(1, D), W2)

    s2 = _scatter_kernel(g2, src2d, dst2d)

    out = pl.pallas_call(
        _tc_out_body,
        grid=(grid,),
        in_specs=[sspec, gspec, vspec, bspec,
                  pl.BlockSpec((D, 1), lambda i: (0, 0)),
                  pl.BlockSpec((1, 1), lambda i: (0, 0))],
        out_specs=vspec,
        out_shape=jax.ShapeDtypeStruct((N_PAD, 1), jnp.float32),
    )(s2, g2, dinv, b2.reshape(1, D), W_out, b_out.reshape(1, 1))

    return out[:N_REAL, 0]


# speedup vs baseline: 11.9031x; 1.0299x over previous
import functools

import jax
import jax.numpy as jnp
from jax import lax
from jax.experimental import pallas as pl
from jax.experimental.pallas import tpu as pltpu
from jax.experimental.pallas import tpu_sc as plsc

N_REAL = 10000
D = 128
E_REAL = 320000

NC = 2
NS = 16
NW = NC * NS
CH = 128

N_PAD = 10240
ROWS_PER_TILE = N_PAD // NS
E_PAD = 2560 * CH
CHUNKS_PER_W = (E_PAD // CH) // NW
E_ALLOC = E_PAD

_mesh = plsc.VectorSubcoreMesh(
    core_axis_name="c", subcore_axis_name="s", num_cores=NC, num_subcores=NS)


def _worker_id():
    cid = lax.axis_index("c")
    sid = lax.axis_index("s")
    return cid, sid, sid * NC + cid


@functools.partial(
    pl.kernel,
    out_type=jax.ShapeDtypeStruct((NC, N_PAD), jnp.float32),
    mesh=_mesh,
    scratch_types=[
        pltpu.VMEM((CHUNKS_PER_W, CH), jnp.int32),
        pltpu.VMEM((CH,), jnp.float32),
        pltpu.VMEM((ROWS_PER_TILE,), jnp.float32),
        pltpu.VMEM_SHARED((N_PAD,), jnp.float32),
        pltpu.SemaphoreType.DMA,
    ],
)
def _deg_kernel(dst_hbm, out_hbm, didx, ones_v, zbuf, hist_sh, sem):
    cid, sid, w = _worker_id()
    one = jnp.full((16,), 1.0, dtype=jnp.float32)
    zero = jnp.zeros((16,), dtype=jnp.float32)
    for k in range(CH // 16):
        ones_v[pl.ds(k * 16, 16)] = one
    for k in range(ROWS_PER_TILE // 16):
        zbuf[pl.ds(k * 16, 16)] = zero
    pltpu.sync_copy(dst_hbm.at[pl.ds(w * CHUNKS_PER_W, CHUNKS_PER_W), :], didx)
    pltpu.sync_copy(zbuf, hist_sh.at[pl.ds(sid * ROWS_PER_TILE, ROWS_PER_TILE)])
    plsc.subcore_barrier()

    K = 8

    def body(g, carry):
        ds_ = [pltpu.async_copy(ones_v, hist_sh.at[didx.at[g * K + b]], sem,
                                add=True) for b in range(K)]
        for d_ in ds_:
            d_.wait()
        return carry

    lax.fori_loop(0, CHUNKS_PER_W // K, body, 0)
    plsc.subcore_barrier()
    sl = pl.ds(sid * ROWS_PER_TILE, ROWS_PER_TILE)
    pltpu.sync_copy(hist_sh.at[sl], zbuf)
    pltpu.sync_copy(zbuf, out_hbm.at[cid, sl])


PASSES0 = 3
PASSES1 = 1
PASS_CH = 40
CPT0 = PASSES0 * PASS_CH
CPT1 = PASSES1 * PASS_CH


@functools.partial(
    pl.kernel,
    out_type=jax.ShapeDtypeStruct((NC, N_PAD, D), jnp.float32),
    mesh=_mesh,
    scratch_types=[
        pltpu.VMEM((PASS_CH, CH), jnp.int32),
        pltpu.VMEM((PASS_CH, CH), jnp.int32),
        pltpu.VMEM((2, CH, D), jnp.float32),
        pltpu.VMEM_SHARED((N_PAD, D), jnp.float32),
        pltpu.SemaphoreType.DMA,
        pltpu.SemaphoreType.DMA,
        pltpu.SemaphoreType.DMA,
        pltpu.SemaphoreType.DMA,
    ],
)
def _scatter_kernel(g_hbm, src_hbm, dst_hbm, out_hbm, sidx, didx, rows,
                    acc_sh, gsem0, gsem1, ssem0, ssem1):
    cid, sid, w = _worker_id()
    gsem = (gsem0, gsem1)
    ssem = (ssem0, ssem1)
    zero = jnp.zeros((16,), dtype=jnp.float32)

    def zbody(i, carry):
        for k in range(D // 16):
            rows[0, i, pl.ds(k * 16, 16)] = zero
        return carry

    lax.fori_loop(0, CH, zbody, 0)
    base = sid * ROWS_PER_TILE
    for r in range(ROWS_PER_TILE // CH):
        pltpu.sync_copy(rows.at[0], acc_sh.at[pl.ds(base + r * CH, CH), :])
    plsc.subcore_barrier()

    def gather(k, b):
        return pltpu.async_copy(g_hbm.at[sidx.at[k]], rows.at[b], gsem[b])

    def gather_wait(k, b):
        pltpu.make_async_copy(g_hbm.at[sidx.at[k]], rows.at[b],
                              gsem[b]).wait()

    def scatter(k, b):
        return pltpu.async_copy(rows.at[b], acc_sh.at[didx.at[k]], ssem[b],
                                add=True)

    def scatter_wait(k, b):
        pltpu.make_async_copy(rows.at[b], acc_sh.at[didx.at[k]],
                              ssem[b]).wait()

    big = cid == 1
    base_ch = jnp.where(big, sid * CPT0, NS * CPT0 + sid * CPT1)

    def run_pass(off):
        pltpu.sync_copy(src_hbm.at[pl.ds(off, PASS_CH), :], sidx)
        pltpu.sync_copy(dst_hbm.at[pl.ds(off, PASS_CH), :], didx)
        gather(0, 0)

        def pair_body(t, carry):
            for u in range(2):
                k = 2 * t + u
                pu, nu = (u + 1) % 2, (u + 1) % 2

                @pl.when(k >= 1)
                def _():
                    scatter_wait(k - 1, pu)

                @pl.when(k + 1 < PASS_CH)
                def _():
                    gather(k + 1, nu)

                gather_wait(k, u)
                scatter(k, u)
            return carry

        lax.fori_loop(0, PASS_CH // 2, pair_body, 0)
        scatter_wait(PASS_CH - 1, 1)

    run_pass(base_ch)
    for p in range(1, max(PASSES0, PASSES1)):
        if p < min(PASSES0, PASSES1):
            run_pass(base_ch + p * PASS_CH)
        else:

            @pl.when(big)
            def _():
                run_pass(base_ch + p * PASS_CH)
    plsc.subcore_barrier()
    outs = []
    for r in range(ROWS_PER_TILE // CH):
        sl = pl.ds(base + r * CH, CH)
        outs.append(pltpu.async_copy(acc_sh.at[sl, :], out_hbm.at[cid, sl, :],
                                     gsem0))
    for d_ in outs:
        d_.wait()


BLK = 1024


def _tc1_body(deg_ref, x_ref, w_ref, g_ref, dinv_ref):
    i = pl.program_id(0)
    deg = deg_ref[:, 0:1] + deg_ref[:, 1:2] + 1.0
    dinv = lax.rsqrt(deg)
    rows = lax.broadcasted_iota(jnp.int32, (BLK, 1), 0) + i * BLK
    dinv = jnp.where(rows < N_REAL, dinv, 0.0)
    g = jnp.dot(x_ref[...], w_ref[...], preferred_element_type=jnp.float32)
    g_ref[...] = g * dinv
    dinv_ref[...] = dinv


def _tc_mid_body(s_ref, g_ref, dinv_ref, b_ref, w_ref, g2_ref):
    s = s_ref[0] + s_ref[1] + g_ref[...]
    h = jnp.maximum(dinv_ref[...] * s + b_ref[...], 0.0)
    g2 = jnp.dot(h, w_ref[...], preferred_element_type=jnp.float32)
    g2_ref[...] = g2 * dinv_ref[...]


def _tc_out_body(s_ref, g_ref, dinv_ref, b_ref, w_ref, bo_ref, o_ref):
    s = s_ref[0] + s_ref[1] + g_ref[...]
    h = jnp.maximum(dinv_ref[...] * s + b_ref[...], 0.0)
    o_ref[...] = jnp.dot(h, w_ref[...],
                         preferred_element_type=jnp.float32) + bo_ref[...]


def _row_specs():
    s = pl.BlockSpec((2, BLK, D), lambda i: (0, i, 0))
    g = pl.BlockSpec((BLK, D), lambda i: (i, 0))
    v = pl.BlockSpec((BLK, 1), lambda i: (i, 0))
    w = pl.BlockSpec((D, D), lambda i: (0, 0))
    b = pl.BlockSpec((1, D), lambda i: (0, 0))
    return s, g, v, w, b


def kernel(x, edge_index, W1, b1, W2, b2, W_out, b_out):
    src = edge_index[0].astype(jnp.int32)
    dst = edge_index[1].astype(jnp.int32)
    pad_node = jnp.int32(N_PAD - 1)
    src2d = jnp.full((E_ALLOC,), pad_node).at[:E_REAL].set(src).reshape(-1, CH)
    dst2d = jnp.full((E_ALLOC,), pad_node).at[:E_REAL].set(dst).reshape(-1, CH)
    x_p = jnp.pad(x, ((0, N_PAD - N_REAL), (0, 0)))

    degp = _deg_kernel(dst2d)
    degp_t = jnp.swapaxes(degp, 0, 1)

    grid = N_PAD // BLK
    sspec, gspec, vspec, wspec, bspec = _row_specs()

    g1, dinv = pl.pallas_call(
        _tc1_body,
        grid=(grid,),
        in_specs=[pl.BlockSpec((BLK, 2), lambda i: (i, 0)), gspec, wspec],
        out_specs=[gspec, vspec],
        out_shape=[jax.ShapeDtypeStruct((N_PAD, D), jnp.float32),
                   jax.ShapeDtypeStruct((N_PAD, 1), jnp.float32)],
    )(degp_t, x_p, W1)

    s1 = _scatter_kernel(g1, src2d, dst2d)

    g2 = pl.pallas_call(
        _tc_mid_body,
        grid=(grid,),
        in_specs=[sspec, gspec, vspec, bspec, wspec],
        out_specs=gspec,
        out_shape=jax.ShapeDtypeStruct((N_PAD, D), jnp.float32),
    )(s1, g1, dinv, b1.reshape(1, D), W2)

    s2 = _scatter_kernel(g2, src2d, dst2d)

    out = pl.pallas_call(
        _tc_out_body,
        grid=(grid,),
        in_specs=[sspec, gspec, vspec, bspec,
                  pl.BlockSpec((D, 1), lambda i: (0, 0)),
                  pl.BlockSpec((1, 1), lambda i: (0, 0))],
        out_specs=vspec,
        out_shape=jax.ShapeDtypeStruct((N_PAD, 1), jnp.float32),
    )(s2, g2, dinv, b2.reshape(1, D), W_out, b_out.reshape(1, 1))

    return out[:N_REAL, 0]
